# baseline (speedup 1.0000x reference)
import functools

import jax
import jax.numpy as jnp
from jax import lax
from jax.experimental import pallas as pl
from jax.experimental.pallas import tpu as pltpu
from jax.experimental.pallas import tpu_sc as plsc

VOCAB = 1000000
EMBED = 64
BATCH = 4096
SEQLEN = 50
N = BATCH * SEQLEN

NC = 2
NS = 16
LANES = 16
NW = NC * NS
PER_W = N // NW

IB = 128
ROWS_PER_W = PER_W // IB
CROWS = 5
CHUNK = CROWS * IB
NCHUNK = ROWS_PER_W // CROWS


def _lane_sum(v):
    perm = lax.iota(jnp.int32, LANES)
    for sh in (8, 4, 2, 1):
        rot = v.at[
            lax.rem(perm + sh, jnp.full((LANES,), LANES, jnp.int32))
        ].get(mode="promise_in_bounds")
        v = v + rot
    return v[0]


def _gather_body(idx_hbm, table_hbm, out_hbm, idx_v, rows_v, sem):
    wid = lax.axis_index("s") * NC + lax.axis_index("c")
    row0 = wid * ROWS_PER_W
    pltpu.sync_copy(idx_hbm.at[wid], idx_v)

    def chunk_body(ci, carry):
        crow = row0 + ci * CROWS
        for j in range(CROWS):
            pltpu.async_copy(
                table_hbm.at[idx_v.at[ci * CROWS + j]],
                rows_v.at[pl.ds(j * IB, IB)],
                sem,
            ).wait()

        def cnt_body(g, acc):
            iv = idx_v[
                ci * CROWS + g // (IB // LANES),
                pl.ds((g % (IB // LANES)) * LANES, LANES),
            ]
            return acc + jnp.where(iv == 0, 1, 0).astype(jnp.int32)

        cnt_vec = lax.fori_loop(
            0, CHUNK // LANES, cnt_body, jnp.zeros((LANES,), jnp.int32)
        )
        cnt = _lane_sum(cnt_vec)

        @pl.when(cnt > 0)
        def _fix():
            def grp_body(g, c2):
                iv = idx_v[
                    ci * CROWS + g // (IB // LANES),
                    pl.ds((g % (IB // LANES)) * LANES, LANES),
                ]
                mv = jnp.where(iv != 0, jnp.float32(1.0), jnp.float32(0.0))
                for rsub in range(LANES):
                    m = jnp.full((LANES,), mv[rsub])
                    r = g * LANES + rsub
                    for c in range(EMBED // LANES):
                        rows_v[r, pl.ds(c * LANES, LANES)] = (
                            rows_v[r, pl.ds(c * LANES, LANES)] * m
                        )
                return c2

            lax.fori_loop(0, CHUNK // LANES, grp_body, 0)

        pltpu.sync_copy(rows_v, out_hbm.at[pl.ds(crow * IB, CHUNK)])
        return carry

    lax.fori_loop(0, NCHUNK, chunk_body, 0)


@jax.jit
def _sc_gather(idx2d, table):
    kern = pl.kernel(
        _gather_body,
        out_type=jax.ShapeDtypeStruct((N, EMBED), jnp.float32),
        mesh=plsc.VectorSubcoreMesh(
            core_axis_name="c", subcore_axis_name="s"
        ),
        scratch_types=[
            pltpu.VMEM((ROWS_PER_W, IB), jnp.int32),
            pltpu.VMEM((CHUNK, EMBED), jnp.float32),
            pltpu.SemaphoreType.DMA,
        ],
        compiler_params=pltpu.CompilerParams(use_tc_tiling_on_sc=False),
    )
    return kern(idx2d, table)


def kernel(x, seq_lengths, table):
    del seq_lengths
    idx3d = x.astype(jnp.int32).reshape(NW, ROWS_PER_W, IB)
    out = _sc_gather(idx3d, table)
    return out.reshape(BATCH, SEQLEN, EMBED)


# speedup vs baseline: 1.0410x; 1.0410x over previous
import jax
import jax.numpy as jnp
from jax import lax
from jax.experimental import pallas as pl
from jax.experimental.pallas import tpu as pltpu
from jax.experimental.pallas import tpu_sc as plsc

VOCAB = 1000000
EMBED = 64
BATCH = 4096
SEQLEN = 50
N = BATCH * SEQLEN

NC = 2
NS = 16
LANES = 16
NW = NC * NS
PER_W = N // NW

IB = 128
ROWS_PER_W = PER_W // IB
CROWS = 5
CHUNK = CROWS * IB
NCHUNK = ROWS_PER_W // CROWS
NGRP = PER_W // LANES


def _lane_sum(v):
    perm = lax.iota(jnp.int32, LANES)
    for sh in (8, 4, 2, 1):
        rot = v.at[
            lax.rem(perm + sh, jnp.full((LANES,), LANES, jnp.int32))
        ].get(mode="promise_in_bounds")
        v = v + rot
    return v[0]


def _idx_group(idx_v, g):
    return idx_v[g // (IB // LANES), pl.ds((g % (IB // LANES)) * LANES, LANES)]


def _gather_body(idx_hbm, table_hbm, out_hbm,
                 idx_v, rows_a, rows_b, zrow_v,
                 gsem_a, gsem_b, osem_a, osem_b):
    wid = lax.axis_index("s") * NC + lax.axis_index("c")
    wbase = wid * PER_W

    pltpu.sync_copy(idx_hbm.at[wid], idx_v)

    rows = (rows_a, rows_b)
    gsems = (gsem_a, gsem_b)
    osems = (osem_a, osem_b)

    def fire(ci):
        buf = rows[ci % 2]
        return [
            pltpu.async_copy(
                table_hbm.at[idx_v.at[ci * CROWS + j]],
                buf.at[pl.ds(j * IB, IB)],
                gsems[ci % 2],
            )
            for j in range(CROWS)
        ]

    gh = {0: fire(0)}

    def det_body(g, acc):
        return jnp.minimum(acc, _idx_group(idx_v, g))

    accmin = lax.fori_loop(
        0, NGRP, det_body, jnp.full((LANES,), VOCAB, jnp.int32), unroll=8
    )
    npad = _lane_sum(jnp.where(accmin == 0, 1, 0).astype(jnp.int32))

    for c in range(EMBED // LANES):
        zrow_v[pl.ds(c * LANES, LANES)] = jnp.zeros((LANES,), jnp.float32)

    oh = {}
    for ci in range(NCHUNK):
        if ci + 1 < NCHUNK:
            if ci - 1 >= 0:
                oh[ci - 1].wait()
            gh[ci + 1] = fire(ci + 1)
        for h in gh[ci]:
            h.wait()
        oh[ci] = pltpu.async_copy(
            rows[ci % 2],
            out_hbm.at[pl.ds(wbase + ci * CHUNK, CHUNK)],
            osems[ci % 2],
        )
    oh[NCHUNK - 2].wait()
    oh[NCHUNK - 1].wait()

    @pl.when(npad > 0)
    def _fix():
        def fix_group(g, c2):
            iv = _idx_group(idx_v, g)
            zc = _lane_sum(jnp.where(iv == 0, 1, 0).astype(jnp.int32))

            @pl.when(zc > 0)
            def _grp():
                for l in range(LANES):
                    @pl.when(iv[l] == 0)
                    def _row():
                        pltpu.sync_copy(
                            zrow_v, out_hbm.at[wbase + g * LANES + l]
                        )
            return c2

        lax.fori_loop(0, NGRP, fix_group, 0)


@jax.jit
def _sc_gather(idx3d, table):
    kern = pl.kernel(
        _gather_body,
        out_type=jax.ShapeDtypeStruct((N, EMBED), jnp.float32),
        mesh=plsc.VectorSubcoreMesh(
            core_axis_name="c", subcore_axis_name="s"
        ),
        scratch_types=[
            pltpu.VMEM((ROWS_PER_W, IB), jnp.int32),
            pltpu.VMEM((CHUNK, EMBED), jnp.float32),
            pltpu.VMEM((CHUNK, EMBED), jnp.float32),
            pltpu.VMEM((EMBED,), jnp.float32),
            pltpu.SemaphoreType.DMA,
            pltpu.SemaphoreType.DMA,
            pltpu.SemaphoreType.DMA,
            pltpu.SemaphoreType.DMA,
        ],
        compiler_params=pltpu.CompilerParams(use_tc_tiling_on_sc=False),
    )
    return kern(idx3d, table)


def kernel(x, seq_lengths, table):
    del seq_lengths
    idx3d = x.astype(jnp.int32).reshape(NW, ROWS_PER_W, IB)
    out = _sc_gather(idx3d, table)
    return out.reshape(BATCH, SEQLEN, EMBED)
